# Initial kernel scaffold; baseline (speedup 1.0000x reference)
import jax
import jax.numpy as jnp
from jax.experimental import pallas as pl


def kernel(scores, bbox_deltas, im_info, anchors):
    raise NotImplementedError("write your pallas kernel here")


# speedup vs baseline: 49.6337x; 49.6337x over previous
import functools

import jax
import jax.numpy as jnp
from jax.experimental import pallas as pl
from jax.experimental.pallas import tpu as pltpu

FEAT_STRIDE = 16.0
PRE_NMS_TOPN = 2000
POST_NMS_TOPN = 300
NMS_THRESH = 0.7
MIN_SIZE = 16.0

_N = 2048
_BLK = 128
_OUT_ROWS = 384
_NEG = -1e9


def _transform_body(fg_ref, dl_ref, anch_ref, im_ref, x1_ref, y1_ref, x2_ref, y2_ref, sc_ref):
    hw = jax.lax.broadcasted_iota(jnp.int32, (9, 2500), 1).astype(jnp.float32)
    row = jnp.floor((hw + 0.5) * (1.0 / 50.0))
    sy = row * FEAT_STRIDE
    sx = (hw - 50.0 * row) * FEAT_STRIDE

    ax1 = anch_ref[:, 0:1] + sx
    ay1 = anch_ref[:, 1:2] + sy
    ax2 = anch_ref[:, 2:3] + sx
    ay2 = anch_ref[:, 3:4] + sy

    widths = ax2 - ax1 + 1.0
    heights = ay2 - ay1 + 1.0
    ctr_x = ax1 + 0.5 * widths
    ctr_y = ay1 + 0.5 * heights

    dx = dl_ref[:, 0, :]
    dy = dl_ref[:, 1, :]
    dw = dl_ref[:, 2, :]
    dh = dl_ref[:, 3, :]

    pred_ctr_x = dx * widths + ctr_x
    pred_ctr_y = dy * heights + ctr_y
    pred_w = jnp.exp(dw) * widths
    pred_h = jnp.exp(dh) * heights

    im_h = im_ref[0:1, 0:1]
    im_w = im_ref[0:1, 1:2]
    im_scale = im_ref[0:1, 2:3]

    x1 = jnp.clip(pred_ctr_x - 0.5 * pred_w, 0.0, im_w - 1.0)
    y1 = jnp.clip(pred_ctr_y - 0.5 * pred_h, 0.0, im_h - 1.0)
    x2 = jnp.clip(pred_ctr_x + 0.5 * pred_w, 0.0, im_w - 1.0)
    y2 = jnp.clip(pred_ctr_y + 0.5 * pred_h, 0.0, im_h - 1.0)

    ws = x2 - x1 + 1.0
    hs = y2 - y1 + 1.0
    min_size = MIN_SIZE * im_scale
    valid = (ws >= min_size) & (hs >= min_size)

    x1_ref[...] = x1
    y1_ref[...] = y1
    x2_ref[...] = x2
    y2_ref[...] = y2
    sc_ref[...] = jnp.where(valid, fg_ref[...], _NEG)


def _transform(fg, dl, anchors, im_info):
    out = jax.ShapeDtypeStruct((9, 2500), jnp.float32)
    return pl.pallas_call(
        _transform_body,
        out_shape=(out, out, out, out, out),
    )(fg, dl, anchors, im_info)


def _nms_body(rows_ref, cols_ref, out_ref, s_mat, lt_mat):
    x1r = rows_ref[0:1, :]
    y1r = rows_ref[1:2, :]
    x2r = rows_ref[2:3, :]
    y2r = rows_ref[3:4, :]
    area_r = (x2r - x1r + 1.0) * (y2r - y1r + 1.0)

    for b in range(_N // _BLK):
        sl = pl.ds(b * _BLK, _BLK)
        x1c = cols_ref[sl, 0:1]
        y1c = cols_ref[sl, 1:2]
        x2c = cols_ref[sl, 2:3]
        y2c = cols_ref[sl, 3:4]
        area_c = (x2c - x1c + 1.0) * (y2c - y1c + 1.0)
        xx1 = jnp.maximum(x1c, x1r)
        yy1 = jnp.maximum(y1c, y1r)
        xx2 = jnp.minimum(x2c, x2r)
        yy2 = jnp.minimum(y2c, y2r)
        inter = jnp.maximum(xx2 - xx1 + 1.0, 0.0) * jnp.maximum(yy2 - yy1 + 1.0, 0.0)
        iou = inter / (area_c + area_r - inter)
        jg = b * _BLK + jax.lax.broadcasted_iota(jnp.int32, (_BLK, _N), 0)
        ig = jax.lax.broadcasted_iota(jnp.int32, (_BLK, _N), 1)
        sup = (iou > NMS_THRESH) & (jg < ig) & (ig < PRE_NMS_TOPN) & (jg < PRE_NMS_TOPN)
        s_mat[sl, :] = sup.astype(jnp.bfloat16)
        lt_mat[sl, :] = (jg <= ig).astype(jnp.bfloat16)

    icol = jax.lax.broadcasted_iota(jnp.int32, (8, _N), 1)
    inb = (icol < PRE_NMS_TOPN).astype(jnp.float32)
    keep0 = inb

    def cond(carry):
        _, changed, it = carry
        return changed & (it < _N)

    def body(carry):
        keep, _, it = carry
        sup = jnp.dot(keep.astype(jnp.bfloat16), s_mat[...],
                      preferred_element_type=jnp.float32)
        nk = jnp.where((sup < 0.5) & (icol < PRE_NMS_TOPN), 1.0, 0.0)
        changed = jnp.sum(jnp.abs(nk - keep)) > 0.0
        return nk, changed, it + 1

    keep, _, _ = jax.lax.while_loop(cond, body, (keep0, True, 0))

    kcount = jnp.sum(keep[0:1, :])
    fill = (1.0 - keep) * inb
    cumk = jnp.dot(keep.astype(jnp.bfloat16), lt_mat[...],
                   preferred_element_type=jnp.float32)
    cumf = jnp.dot(fill.astype(jnp.bfloat16), lt_mat[...],
                   preferred_element_type=jnp.float32)
    pos = jnp.where(keep > 0.5, cumk - 1.0, kcount + cumf - 1.0)
    pos = jnp.where(inb > 0.5, pos, 1e9)
    pos1 = pos[0:1, :]

    riota = jax.lax.broadcasted_iota(jnp.int32, (_OUT_ROWS, _N), 0).astype(jnp.float32)
    posb = jnp.broadcast_to(pos1, (_OUT_ROWS, _N))
    sel = riota == posb

    def pick(vals_row):
        v = jnp.broadcast_to(vals_row, (_OUT_ROWS, _N))
        return jnp.max(jnp.where(sel, v, -3.0e38), axis=1, keepdims=True)

    rvec = jax.lax.broadcasted_iota(jnp.int32, (_OUT_ROWS, 1), 0).astype(jnp.float32)
    out_ref[:, 0:1] = jnp.zeros((_OUT_ROWS, 1), jnp.float32)
    out_ref[:, 1:2] = pick(x1r)
    out_ref[:, 2:3] = pick(y1r)
    out_ref[:, 3:4] = pick(x2r)
    out_ref[:, 4:5] = pick(y2r)
    scpick = pick(rows_ref[4:5, :])
    out_ref[:, 5:6] = jnp.where(rvec < kcount, scpick, _NEG)
    out_ref[:, 6:8] = jnp.zeros((_OUT_ROWS, 2), jnp.float32)


def _nms(rows, cols):
    return pl.pallas_call(
        _nms_body,
        out_shape=jax.ShapeDtypeStruct((_OUT_ROWS, 8), jnp.float32),
        scratch_shapes=[
            pltpu.VMEM((_N, _N), jnp.bfloat16),
            pltpu.VMEM((_N, _N), jnp.bfloat16),
        ],
    )(rows, cols)


def kernel(scores, bbox_deltas, im_info, anchors):
    A = anchors.shape[0]
    H, W = scores.shape[2], scores.shape[3]
    fg = scores[0, A:].reshape(A, H * W)
    dl = bbox_deltas[0].reshape(A, 4, H * W)

    x1, y1, x2, y2, scm = _transform(fg, dl, anchors, im_info)

    def flat(v):
        return v.T.reshape(-1)

    scf = flat(scm)
    top_scores, top_idx = jax.lax.top_k(scf, PRE_NMS_TOPN)
    bx1 = flat(x1)[top_idx]
    by1 = flat(y1)[top_idx]
    bx2 = flat(x2)[top_idx]
    by2 = flat(y2)[top_idx]

    pad = _N - PRE_NMS_TOPN
    z = jnp.zeros((pad,), jnp.float32)
    rows = jnp.stack([
        jnp.concatenate([bx1, z]),
        jnp.concatenate([by1, z]),
        jnp.concatenate([bx2, z]),
        jnp.concatenate([by2, z]),
        jnp.concatenate([top_scores, z]),
        jnp.zeros((_N,), jnp.float32),
        jnp.zeros((_N,), jnp.float32),
        jnp.zeros((_N,), jnp.float32),
    ])
    cols = rows.T

    out = _nms(rows, cols)
    return out[:POST_NMS_TOPN, :6]
